# Initial kernel scaffold; baseline (speedup 1.0000x reference)
import functools

import jax
import jax.numpy as jnp
from jax import lax
from jax.experimental import pallas as pl
from jax.experimental.pallas import tpu as pltpu
from jax.experimental.pallas import tpu_sc as plsc

VOCAB = 100000
UNKNOWN_IDX = 1

_info = plsc.get_sparse_core_info()
_NC, _NS, _L = _info.num_cores, _info.num_subcores, _info.num_lanes
_NW = _NC * _NS


def _make_kernel(n_total: int):
    chunk = n_total // _NW
    assert chunk % _L == 0 and chunk % 8 == 0
    mesh = plsc.VectorSubcoreMesh(core_axis_name="c", subcore_axis_name="s")

    @functools.partial(
        pl.kernel,
        mesh=mesh,
        out_type=jax.ShapeDtypeStruct((n_total,), jnp.int32),
        scratch_types=[
            pltpu.VMEM((VOCAB,), jnp.int32),
            pltpu.VMEM((chunk,), jnp.int32),
            pltpu.VMEM((chunk,), jnp.int32),
            pltpu.SemaphoreType.DMA,
            pltpu.SemaphoreType.DMA,
        ],
    )
    def k(x_hbm, lookup_hbm, out_hbm, table_v, xin_v, out_v, sem_t, sem_x):
        wid = lax.axis_index("s") * _NC + lax.axis_index("c")
        base = wid * chunk
        cp_t = pltpu.async_copy(lookup_hbm, table_v, sem_t)
        cp_x = pltpu.async_copy(x_hbm.at[pl.ds(base, chunk)], xin_v, sem_x)
        cp_x.wait()
        cp_t.wait()

        def body(i, carry):
            off = i * _L
            idx = xin_v[pl.ds(off, _L)]
            mask = (idx >= 0) & (idx < VOCAB)
            safe = jnp.clip(idx, 0, VOCAB - 1)
            g = plsc.load_gather(table_v, [safe])
            out_v[pl.ds(off, _L)] = jnp.where(mask, g + UNKNOWN_IDX,
                                              jnp.int32(UNKNOWN_IDX))
            return carry

        lax.fori_loop(0, chunk // _L, body, 0)
        pltpu.sync_copy(out_v, out_hbm.at[pl.ds(base, chunk)])

    return k


def kernel(x, lookup):
    shape = x.shape
    n = x.size
    flat = jnp.reshape(x, (n,))
    out = _make_kernel(n)(flat, lookup)
    return jnp.reshape(out, shape)


# speedup vs baseline: 67.6109x; 67.6109x over previous
import functools

import jax
import jax.numpy as jnp
from jax import lax
from jax.experimental import pallas as pl
from jax.experimental.pallas import tpu as pltpu
from jax.experimental.pallas import tpu_sc as plsc

VOCAB = 100000
UNKNOWN_IDX = 1

_NC, _NS, _L = 2, 16, 16
_NW = _NC * _NS


def _make_kernel(n_total: int):
    chunk = n_total // _NW
    assert chunk % _L == 0 and chunk % 8 == 0
    mesh = plsc.VectorSubcoreMesh(core_axis_name="c", subcore_axis_name="s")

    @functools.partial(
        pl.kernel,
        mesh=mesh,
        out_type=jax.ShapeDtypeStruct((n_total,), jnp.int32),
        scratch_types=[
            pltpu.VMEM((VOCAB,), jnp.int32),
            pltpu.VMEM((chunk,), jnp.int32),
            pltpu.VMEM((chunk,), jnp.int32),
            pltpu.SemaphoreType.DMA,
            pltpu.SemaphoreType.DMA,
        ],
        compiler_params=pltpu.CompilerParams(needs_layout_passes=False),
    )
    def k(x_hbm, lookup_hbm, out_hbm, table_v, xin_v, out_v, sem_t, sem_x):
        wid = lax.axis_index("s") * _NC + lax.axis_index("c")
        base = wid * chunk
        cp_t = pltpu.async_copy(lookup_hbm, table_v, sem_t)
        cp_x = pltpu.async_copy(x_hbm.at[pl.ds(base, chunk)], xin_v, sem_x)
        cp_x.wait()
        cp_t.wait()

        def body(i, carry):
            off = i * _L
            idx = xin_v[pl.ds(off, _L)]
            mask = (idx >= 0) & (idx < VOCAB)
            safe = jnp.clip(idx, 0, VOCAB - 1)
            g = plsc.load_gather(table_v, [safe])
            out_v[pl.ds(off, _L)] = jnp.where(mask, g + UNKNOWN_IDX,
                                              jnp.int32(UNKNOWN_IDX))
            return carry

        lax.fori_loop(0, chunk // _L, body, 0)
        pltpu.sync_copy(out_v, out_hbm.at[pl.ds(base, chunk)])

    return k


def kernel(x, lookup):
    shape = x.shape
    n = x.size
    flat = jnp.reshape(x, (n,))
    out = _make_kernel(n)(flat, lookup)
    return jnp.reshape(out, shape)


# speedup vs baseline: 72.3702x; 1.0704x over previous
import functools

import jax
import jax.numpy as jnp
from jax import lax
from jax.experimental import pallas as pl
from jax.experimental.pallas import tpu as pltpu
from jax.experimental.pallas import tpu_sc as plsc

VOCAB = 100000
UNKNOWN_IDX = 1

_NC, _NS, _L = 2, 16, 16
_NW = _NC * _NS


def _make_kernel(n_total: int):
    chunk = n_total // _NW
    assert chunk % _L == 0 and chunk % 8 == 0
    mesh = plsc.VectorSubcoreMesh(core_axis_name="c", subcore_axis_name="s")

    @functools.partial(
        pl.kernel,
        mesh=mesh,
        out_type=jax.ShapeDtypeStruct((n_total,), jnp.int32),
        scratch_types=[
            pltpu.VMEM((VOCAB,), jnp.int32),
            pltpu.VMEM((chunk,), jnp.int32),
            pltpu.VMEM((chunk,), jnp.int32),
            pltpu.SemaphoreType.DMA,
            pltpu.SemaphoreType.DMA,
        ],
        compiler_params=pltpu.CompilerParams(needs_layout_passes=False),
    )
    def k(x_hbm, lookup_hbm, out_hbm, table_v, xin_v, out_v, sem_t, sem_x):
        wid = lax.axis_index("s") * _NC + lax.axis_index("c")
        base = wid * chunk
        cp_t = pltpu.async_copy(lookup_hbm, table_v, sem_t)
        cp_x = pltpu.async_copy(x_hbm.at[pl.ds(base, chunk)], xin_v, sem_x)
        cp_x.wait()
        cp_t.wait()

        @plsc.parallel_loop(0, chunk, _L, unroll=8)
        def body(off):
            idx = xin_v[pl.ds(off, _L)]
            mask = plsc.bitcast(idx, jnp.uint32) < jnp.uint32(VOCAB)
            safe = jnp.where(mask, idx, jnp.int32(0))
            g = plsc.load_gather(table_v, [safe])
            out_v[pl.ds(off, _L)] = jnp.where(mask, g + UNKNOWN_IDX,
                                              jnp.int32(UNKNOWN_IDX))
        pltpu.sync_copy(out_v, out_hbm.at[pl.ds(base, chunk)])

    return k


def kernel(x, lookup):
    shape = x.shape
    n = x.size
    flat = jnp.reshape(x, (n,))
    out = _make_kernel(n)(flat, lookup)
    return jnp.reshape(out, shape)


# speedup vs baseline: 86.0661x; 1.1892x over previous
import functools

import jax
import jax.numpy as jnp
from jax import lax
from jax.experimental import pallas as pl
from jax.experimental.pallas import tpu as pltpu
from jax.experimental.pallas import tpu_sc as plsc

VOCAB = 100000
UNKNOWN_IDX = 1

_NC, _NS, _L = 2, 16, 16
_NW = _NC * _NS


def _make_kernel(rows: int, cols: int):
    r_per = rows // _NW
    n_full = cols // _L
    tail = cols - n_full * _L
    if tail:
        assert _L % tail == 0 and r_per % (_L // tail) == 0
        rows_per_group = _L // tail
    mesh = plsc.VectorSubcoreMesh(core_axis_name="c", subcore_axis_name="s")

    @functools.partial(
        pl.kernel,
        mesh=mesh,
        out_type=jax.ShapeDtypeStruct((rows, cols), jnp.int32),
        scratch_types=[
            pltpu.VMEM((VOCAB,), jnp.int32),
            pltpu.VMEM((r_per, cols), jnp.int32),
            pltpu.SemaphoreType.DMA,
            pltpu.SemaphoreType.DMA,
        ],
        compiler_params=pltpu.CompilerParams(needs_layout_passes=False),
    )
    def k(x_hbm, lookup_hbm, out_hbm, table_v, buf_v, sem_t, sem_x):
        wid = lax.axis_index("s") * _NC + lax.axis_index("c")
        base = wid * r_per
        cp_t = pltpu.async_copy(lookup_hbm, table_v, sem_t)
        cp_x = pltpu.async_copy(x_hbm.at[pl.ds(base, r_per)], buf_v, sem_x)
        cp_x.wait()
        cp_t.wait()

        def transform(idx):
            mask = plsc.bitcast(idx, jnp.uint32) < jnp.uint32(VOCAB)
            safe = jnp.where(mask, idx, jnp.int32(0))
            g = plsc.load_gather(table_v, [safe])
            return jnp.where(mask, g + UNKNOWN_IDX, jnp.int32(UNKNOWN_IDX))

        @plsc.parallel_loop(0, r_per, 1, unroll=2)
        def rowbody(r):
            for j in range(n_full):
                sl = pl.ds(j * _L, _L)
                buf_v[r, sl] = transform(buf_v[r, sl])

        if tail:
            @plsc.parallel_loop(0, r_per, rows_per_group, unroll=2)
            def tailbody(r0):
                lanes = lax.iota(jnp.int32, _L)
                row = r0 + (lanes // tail)
                col = (cols - tail) + (lanes % tail)
                vals = plsc.load_gather(buf_v, [row, col])
                plsc.store_scatter(buf_v, [row, col], transform(vals))

        pltpu.sync_copy(buf_v, out_hbm.at[pl.ds(base, r_per)])

    return k


def kernel(x, lookup):
    rows, cols = x.shape
    return _make_kernel(rows, cols)(x, lookup)


# speedup vs baseline: 86.1540x; 1.0010x over previous
import functools

import jax
import jax.numpy as jnp
from jax import lax
from jax.experimental import pallas as pl
from jax.experimental.pallas import tpu as pltpu
from jax.experimental.pallas import tpu_sc as plsc

VOCAB = 100000
UNKNOWN_IDX = 1

_NC, _NS, _L = 2, 16, 16
_NW = _NC * _NS


def _make_kernel(rows: int, cols: int):
    r_per = rows // _NW
    n_full = cols // _L
    tail = cols - n_full * _L
    if tail:
        assert _L % tail == 0 and r_per % (_L // tail) == 0
        rows_per_group = _L // tail
    mesh = plsc.VectorSubcoreMesh(core_axis_name="c", subcore_axis_name="s")

    @functools.partial(
        pl.kernel,
        mesh=mesh,
        out_type=jax.ShapeDtypeStruct((rows, cols), jnp.int32),
        scratch_types=[
            pltpu.VMEM((VOCAB,), jnp.int32),
            pltpu.VMEM((r_per, cols), jnp.int32),
            pltpu.SemaphoreType.DMA,
            pltpu.SemaphoreType.DMA,
        ],
        compiler_params=pltpu.CompilerParams(needs_layout_passes=False),
    )
    def k(x_hbm, lookup_hbm, out_hbm, table_v, buf_v, sem_t, sem_x):
        wid = lax.axis_index("s") * _NC + lax.axis_index("c")
        base = wid * r_per
        n_split = 4
        tchunk = VOCAB // n_split
        cps = [
            pltpu.async_copy(
                lookup_hbm.at[pl.ds(i * tchunk, tchunk)],
                table_v.at[pl.ds(i * tchunk, tchunk)],
                sem_t,
            )
            for i in range(n_split)
        ]
        cp_x = pltpu.async_copy(x_hbm.at[pl.ds(base, r_per)], buf_v, sem_x)
        cp_x.wait()
        for cp in cps:
            cp.wait()

        def transform(idx):
            mask = plsc.bitcast(idx, jnp.uint32) < jnp.uint32(VOCAB)
            safe = jnp.where(mask, idx, jnp.int32(0))
            g = plsc.load_gather(table_v, [safe])
            return jnp.where(mask, g + UNKNOWN_IDX, jnp.int32(UNKNOWN_IDX))

        @plsc.parallel_loop(0, r_per, 1, unroll=2)
        def rowbody(r):
            for j in range(n_full):
                sl = pl.ds(j * _L, _L)
                buf_v[r, sl] = transform(buf_v[r, sl])

        if tail:
            @plsc.parallel_loop(0, r_per, rows_per_group, unroll=2)
            def tailbody(r0):
                lanes = lax.iota(jnp.int32, _L)
                row = r0 + (lanes // tail)
                col = (cols - tail) + (lanes % tail)
                vals = plsc.load_gather(buf_v, [row, col])
                plsc.store_scatter(buf_v, [row, col], transform(vals))

        pltpu.sync_copy(buf_v, out_hbm.at[pl.ds(base, r_per)])

    return k


def kernel(x, lookup):
    rows, cols = x.shape
    return _make_kernel(rows, cols)(x, lookup)


# speedup vs baseline: 88.1689x; 1.0234x over previous
import functools

import jax
import jax.numpy as jnp
from jax import lax
from jax.experimental import pallas as pl
from jax.experimental.pallas import tpu as pltpu
from jax.experimental.pallas import tpu_sc as plsc

VOCAB = 100000
UNKNOWN_IDX = 1

_NC, _NS, _L = 2, 16, 16
_NW = _NC * _NS


def _make_kernel(rows: int, cols: int):
    r_per = rows // _NW
    n_full = cols // _L
    tail = cols - n_full * _L
    if tail:
        assert _L % tail == 0 and r_per % (_L // tail) == 0
        rows_per_group = _L // tail
    mesh = plsc.VectorSubcoreMesh(core_axis_name="c", subcore_axis_name="s")

    @functools.partial(
        pl.kernel,
        mesh=mesh,
        out_type=jax.ShapeDtypeStruct((rows, cols), jnp.int32),
        scratch_types=[
            pltpu.VMEM((VOCAB,), jnp.int32),
            pltpu.VMEM((r_per, cols), jnp.int32),
            pltpu.SemaphoreType.DMA,
            pltpu.SemaphoreType.DMA,
        ],
        compiler_params=pltpu.CompilerParams(needs_layout_passes=False),
    )
    def k(x_hbm, lookup_hbm, out_hbm, table_v, buf_v, sem_t, sem_x):
        wid = lax.axis_index("s") * _NC + lax.axis_index("c")
        base = wid * r_per
        n_split = 20
        tchunk = VOCAB // n_split
        cps = []
        for i in range(n_split):
            off = ((wid + i) % n_split) * tchunk
            cps.append(pltpu.async_copy(
                lookup_hbm.at[pl.ds(off, tchunk)],
                table_v.at[pl.ds(off, tchunk)],
                sem_t,
            ))
        cp_x = pltpu.async_copy(x_hbm.at[pl.ds(base, r_per)], buf_v, sem_x)
        cp_x.wait()
        for cp in cps:
            cp.wait()

        def transform(idx):
            mask = plsc.bitcast(idx, jnp.uint32) < jnp.uint32(VOCAB)
            safe = jnp.where(mask, idx, jnp.int32(0))
            g = plsc.load_gather(table_v, [safe])
            return jnp.where(mask, g + UNKNOWN_IDX, jnp.int32(UNKNOWN_IDX))

        @plsc.parallel_loop(0, r_per, 1, unroll=2)
        def rowbody(r):
            for j in range(n_full):
                sl = pl.ds(j * _L, _L)
                buf_v[r, sl] = transform(buf_v[r, sl])

        if tail:
            @plsc.parallel_loop(0, r_per, rows_per_group, unroll=2)
            def tailbody(r0):
                lanes = lax.iota(jnp.int32, _L)
                row = r0 + (lanes // tail)
                col = (cols - tail) + (lanes % tail)
                vals = plsc.load_gather(buf_v, [row, col])
                plsc.store_scatter(buf_v, [row, col], transform(vals))

        pltpu.sync_copy(buf_v, out_hbm.at[pl.ds(base, r_per)])

    return k


def kernel(x, lookup):
    rows, cols = x.shape
    return _make_kernel(rows, cols)(x, lookup)


# speedup vs baseline: 102.9999x; 1.1682x over previous
import functools

import jax
import jax.numpy as jnp
from jax import lax
from jax.experimental import pallas as pl
from jax.experimental.pallas import tpu as pltpu
from jax.experimental.pallas import tpu_sc as plsc

VOCAB = 100000
UNKNOWN_IDX = 1

_NC, _NS, _L = 2, 16, 16
_NW = _NC * _NS


def _make_kernel(rows: int, cols: int):
    r_per = rows // _NW
    n_full = cols // _L
    tail = cols - n_full * _L
    if tail:
        assert _L % tail == 0 and r_per % (_L // tail) == 0
        rows_per_group = _L // tail
    mesh = plsc.VectorSubcoreMesh(core_axis_name="c", subcore_axis_name="s")

    @functools.partial(
        pl.kernel,
        mesh=mesh,
        out_type=jax.ShapeDtypeStruct((rows, cols), jnp.int32),
        scratch_types=[
            pltpu.VMEM((VOCAB,), jnp.int32),
            pltpu.VMEM((r_per, cols), jnp.int32),
            pltpu.VMEM_SHARED((VOCAB,), jnp.int32),
            pltpu.SemaphoreType.DMA,
            pltpu.SemaphoreType.DMA,
            pltpu.SemaphoreType.DMA,
        ],
        compiler_params=pltpu.CompilerParams(needs_layout_passes=False),
    )
    def k(x_hbm, lookup_hbm, out_hbm, table_v, buf_v, sp_table, sem_t,
          sem_x, sem_s):
        sid = lax.axis_index("s")
        wid = sid * _NC + lax.axis_index("c")
        base = wid * r_per
        cp_x = pltpu.async_copy(x_hbm.at[pl.ds(base, r_per)], buf_v, sem_x)

        sl_a = 6256
        sl_b = VOCAB - 15 * sl_a

        @pl.when(sid < 15)
        def _():
            pltpu.async_copy(
                lookup_hbm.at[pl.ds(sid * sl_a, sl_a)],
                table_v.at[pl.ds(sid * sl_a, sl_a)],
                sem_s,
            ).wait()
            pltpu.async_copy(
                table_v.at[pl.ds(sid * sl_a, sl_a)],
                sp_table.at[pl.ds(sid * sl_a, sl_a)],
                sem_s,
            ).wait()

        @pl.when(sid == 15)
        def _():
            pltpu.async_copy(
                lookup_hbm.at[pl.ds(15 * sl_a, sl_b)],
                table_v.at[pl.ds(15 * sl_a, sl_b)],
                sem_s,
            ).wait()
            pltpu.async_copy(
                table_v.at[pl.ds(15 * sl_a, sl_b)],
                sp_table.at[pl.ds(15 * sl_a, sl_b)],
                sem_s,
            ).wait()

        plsc.subcore_barrier()
        cp_t = pltpu.async_copy(sp_table, table_v, sem_t)
        cp_x.wait()
        cp_t.wait()

        def transform(idx):
            mask = plsc.bitcast(idx, jnp.uint32) < jnp.uint32(VOCAB)
            safe = jnp.where(mask, idx, jnp.int32(0))
            g = plsc.load_gather(table_v, [safe])
            return jnp.where(mask, g + UNKNOWN_IDX, jnp.int32(UNKNOWN_IDX))

        @plsc.parallel_loop(0, r_per, 1, unroll=2)
        def rowbody(r):
            for j in range(n_full):
                sl = pl.ds(j * _L, _L)
                buf_v[r, sl] = transform(buf_v[r, sl])

        if tail:
            @plsc.parallel_loop(0, r_per, rows_per_group, unroll=2)
            def tailbody(r0):
                lanes = lax.iota(jnp.int32, _L)
                row = r0 + (lanes // tail)
                col = (cols - tail) + (lanes % tail)
                vals = plsc.load_gather(buf_v, [row, col])
                plsc.store_scatter(buf_v, [row, col], transform(vals))

        pltpu.sync_copy(buf_v, out_hbm.at[pl.ds(base, r_per)])

    return k


def kernel(x, lookup):
    rows, cols = x.shape
    return _make_kernel(rows, cols)(x, lookup)
